# baseline (speedup 1.0000x reference)
import functools

import jax
import jax.numpy as jnp
from jax import lax
from jax.experimental import pallas as pl
from jax.experimental.pallas import tpu as pltpu
from jax.experimental.pallas import tpu_sc as plsc

B, C, N, S = 8, 128, 4096, 64
L = 16
NCHUNK = N // L
NSEGV = S // L
ROWS_PER_W = (B * C) // 32
RPC = 8
NDMA = ROWS_PER_W // RPC


def _sc_body(x_hbm, idx_hbm, cnt_hbm, out_hbm,
             idxbuf, xbufa, xbufb, scanbuf, ccbuf, ebuf, gebuf, gqbuf,
             invbuf, gbidx, tstage, tpfull, outstage, sema, semb):
    cid = lax.axis_index("c")
    sid = lax.axis_index("s")
    wid = sid * 2 + cid
    b = wid // 4
    c0 = (wid % 4) * ROWS_PER_W
    rowbase0 = (b * C + c0) * N

    iota = lax.iota(jnp.int32, L)
    xbufs = [xbufa, xbufb]
    sems = [sema, semb]

    cps = [pltpu.async_copy(
        x_hbm.at[pl.ds(rowbase0, RPC * N)], xbufa, sema)]

    pltpu.sync_copy(idx_hbm.at[pl.ds(b * N, N)], idxbuf.at[pl.ds(0, N)])
    pltpu.sync_copy(cnt_hbm.at[pl.ds(b * S, S)], invbuf)
    idxbuf[pl.ds(N, L)] = jnp.full((L,), S, jnp.int32)
    scanbuf[pl.ds(N, L)] = jnp.zeros((L,), jnp.float32)
    ccbuf[pl.ds(NCHUNK, L)] = jnp.zeros((L,), jnp.float32)
    tstage[pl.ds(ROWS_PER_W * S, L)] = jnp.zeros((L,), jnp.float32)
    for i in range(NSEGV):
        ebuf[pl.ds(i * L, L)] = jnp.zeros((L,), jnp.int32)
        invbuf[pl.ds(i * L, L)] = 1.0 / invbuf[pl.ds(i * L, L)]

    @plsc.parallel_loop(0, NCHUNK, 1, unroll=4)
    def bmeta(q):
        base = q * L
        a = idxbuf[pl.ds(base, L)]
        nxt = plsc.load_gather(idxbuf, [iota + (base + 1)])
        pos = iota + (base + 1)
        plsc.store_scatter(ebuf, [a], pos, mask=a != nxt)

    carry = jnp.zeros((L,), jnp.int32)
    for i in range(NSEGV):
        m = jnp.maximum(plsc.cummax(ebuf[pl.ds(i * L, L)]), carry)
        gebuf[pl.ds(i * L, L)] = jnp.where(m >= 1, m - 1, N)
        gqbuf[pl.ds(i * L, L)] = jnp.where(m >= 1, (m - 1) >> 4, NCHUNK)
        carry = jnp.full((L,), jnp.max(m), jnp.int32)

    for i in range(NCHUNK // L):
        gbidx[pl.ds(i * L, L)] = (iota + i * L) * L + (L - 1)

    @plsc.parallel_loop(0, ROWS_PER_W * NSEGV, 1, unroll=4)
    def buildtp(k):
        glob = iota + k * L
        tpfull[pl.ds(k * L, L)] = jnp.where(
            (glob & (S - 1)) >= 1, glob - 1, ROWS_PER_W * S)

    ge_vs = [gebuf[pl.ds(i * L, L)] for i in range(NSEGV)]
    gq_vs = [gqbuf[pl.ds(i * L, L)] for i in range(NSEGV)]

    for g in range(NDMA):
        if g + 1 < NDMA:
            cps.append(pltpu.async_copy(
                x_hbm.at[pl.ds(rowbase0 + (g + 1) * RPC * N, RPC * N)],
                xbufs[(g + 1) % 2], sems[(g + 1) % 2]))
        cps[g].wait()
        xbuf = xbufs[g % 2]

        def row(r, carry_r):
            rbase = r * N

            @plsc.parallel_loop(0, NCHUNK, 1, unroll=128)
            def passa(q):
                s = plsc.cumsum(xbuf[pl.ds(rbase + q * L, L)])
                scanbuf[pl.ds(q * L, L)] = s

            @plsc.parallel_loop(0, NCHUNK // L, 1, unroll=16,
                                carry=jnp.zeros((L,), jnp.float32))
            def passb(i, carry_b):
                gsum = plsc.load_gather(scanbuf, [gbidx[pl.ds(i * L, L)]])
                inc = plsc.cumsum(gsum)
                ccbuf[pl.ds(i * L, L)] = inc - gsum + carry_b
                return carry_b + jnp.full((L,), jnp.sum(gsum), jnp.float32)

            obase = (g * RPC + r) * S
            for i in range(NSEGV):
                t = (plsc.load_gather(scanbuf, [ge_vs[i]])
                     + plsc.load_gather(ccbuf, [gq_vs[i]]))
                tstage[pl.ds(obase + i * L, L)] = t
            return carry_r

        lax.fori_loop(0, RPC, row, 0)

    @plsc.parallel_loop(0, ROWS_PER_W * NSEGV, 1, unroll=8)
    def passc(k):
        t = tstage[pl.ds(k * L, L)]
        tp = plsc.load_gather(tstage, [tpfull[pl.ds(k * L, L)]])
        iv = invbuf[pl.ds((k & (NSEGV - 1)) * L, L)]
        outstage[pl.ds(k * L, L)] = (t - tp) * iv

    pltpu.sync_copy(outstage,
                    out_hbm.at[pl.ds((b * C + c0) * S, ROWS_PER_W * S)])


@jax.jit
def _run(x, idx, cnt):
    mesh = plsc.VectorSubcoreMesh(
        core_axis_name="c", subcore_axis_name="s", num_cores=2,
        num_subcores=16)
    f = pl.kernel(
        _sc_body,
        out_type=jax.ShapeDtypeStruct((B * C * S,), jnp.float32),
        mesh=mesh,
        compiler_params=pltpu.CompilerParams(
            needs_layout_passes=False,
            skip_device_barrier=True,
            disable_bounds_checks=True,
            disable_semaphore_checks=True),
        scratch_types=[
            pltpu.VMEM((N + L,), jnp.int32),
            pltpu.VMEM((RPC * N,), jnp.float32),
            pltpu.VMEM((RPC * N,), jnp.float32),
            pltpu.VMEM((N + L,), jnp.float32),
            pltpu.VMEM((NCHUNK + L,), jnp.float32),
            pltpu.VMEM((S,), jnp.int32),
            pltpu.VMEM((S,), jnp.int32),
            pltpu.VMEM((S,), jnp.int32),
            pltpu.VMEM((S,), jnp.float32),
            pltpu.VMEM((NCHUNK,), jnp.int32),
            pltpu.VMEM((ROWS_PER_W * S + L,), jnp.float32),
            pltpu.VMEM((ROWS_PER_W * S,), jnp.int32),
            pltpu.VMEM((ROWS_PER_W * S,), jnp.float32),
            pltpu.SemaphoreType.DMA,
            pltpu.SemaphoreType.DMA,
        ],
    )
    return f(x, idx, cnt)


def kernel(input, slice_idx_mat, slice_counts):
    x = input.reshape(B * C * N)
    idx = slice_idx_mat.astype(jnp.int32).reshape(B * N)
    cnt = slice_counts.reshape(B * S).astype(jnp.float32)
    out = _run(x, idx, cnt)
    return out.reshape(B, C, S, 1)


# speedup vs baseline: 1.0244x; 1.0244x over previous
import functools

import jax
import jax.numpy as jnp
from jax import lax
from jax.experimental import pallas as pl
from jax.experimental.pallas import tpu as pltpu
from jax.experimental.pallas import tpu_sc as plsc

B, C, N, S = 8, 128, 4096, 64
L = 16
NCHUNK = N // L
NSEGV = S // L
ROWS_PER_W = (B * C) // 32
RPC = 8
NDMA = ROWS_PER_W // RPC


def _sc_body(x_hbm, idx_hbm, cnt_hbm, out_hbm,
             idxbuf, xbufa, xbufb, scanbuf, ccbuf, ebuf, gebuf, gqbuf,
             invbuf, gbidx, tstage, tpfull, outstage, sema, semb):
    cid = lax.axis_index("c")
    sid = lax.axis_index("s")
    wid = sid * 2 + cid
    b = wid // 4
    c0 = (wid % 4) * ROWS_PER_W
    rowbase0 = (b * C + c0) * N

    iota = lax.iota(jnp.int32, L)
    xbufs = [xbufa, xbufb]
    sems = [sema, semb]

    cps = [pltpu.async_copy(
        x_hbm.at[pl.ds(rowbase0, RPC * N)], xbufa, sema)]

    pltpu.sync_copy(idx_hbm.at[pl.ds(b * N, N)], idxbuf.at[pl.ds(0, N)])
    pltpu.sync_copy(cnt_hbm.at[pl.ds(b * S, S)], invbuf)
    idxbuf[pl.ds(N, L)] = jnp.full((L,), S, jnp.int32)
    scanbuf[pl.ds(N, L)] = jnp.zeros((L,), jnp.float32)
    ccbuf[pl.ds(NCHUNK, L)] = jnp.zeros((L,), jnp.float32)
    tstage[pl.ds(ROWS_PER_W * S, L)] = jnp.zeros((L,), jnp.float32)
    for i in range(NSEGV):
        ebuf[pl.ds(i * L, L)] = jnp.zeros((L,), jnp.int32)
        invbuf[pl.ds(i * L, L)] = 1.0 / invbuf[pl.ds(i * L, L)]

    @plsc.parallel_loop(0, NCHUNK, 1, unroll=4)
    def bmeta(q):
        base = q * L
        a = idxbuf[pl.ds(base, L)]
        nxt = plsc.load_gather(idxbuf, [iota + (base + 1)])
        pos = iota + (base + 1)
        plsc.store_scatter(ebuf, [a], pos, mask=a != nxt)

    carry = jnp.zeros((L,), jnp.int32)
    for i in range(NSEGV):
        m = jnp.maximum(plsc.cummax(ebuf[pl.ds(i * L, L)]), carry)
        gebuf[pl.ds(i * L, L)] = jnp.where(m >= 1, m - 1, N)
        gqbuf[pl.ds(i * L, L)] = jnp.where(m >= 1, (m - 1) >> 4, NCHUNK)
        carry = jnp.full((L,), jnp.max(m), jnp.int32)

    for i in range(NCHUNK // L):
        gbidx[pl.ds(i * L, L)] = (iota + i * L) * L + (L - 1)

    @plsc.parallel_loop(0, ROWS_PER_W * NSEGV, 1, unroll=4)
    def buildtp(k):
        glob = iota + k * L
        tpfull[pl.ds(k * L, L)] = jnp.where(
            (glob & (S - 1)) >= 1, glob - 1, ROWS_PER_W * S)

    ge_vs = [gebuf[pl.ds(i * L, L)] for i in range(NSEGV)]
    gq_vs = [gqbuf[pl.ds(i * L, L)] for i in range(NSEGV)]

    for g in range(NDMA):
        if g + 1 < NDMA:
            cps.append(pltpu.async_copy(
                x_hbm.at[pl.ds(rowbase0 + (g + 1) * RPC * N, RPC * N)],
                xbufs[(g + 1) % 2], sems[(g + 1) % 2]))
        cps[g].wait()
        xbuf = xbufs[g % 2]

        def row(r, carry_r):
            rbase = r * N

            @plsc.parallel_loop(0, NCHUNK, 1, unroll=64)
            def passa(q):
                s = plsc.cumsum(xbuf[pl.ds(rbase + q * L, L)])
                scanbuf[pl.ds(q * L, L)] = s

            @plsc.parallel_loop(0, NCHUNK // L, 1, unroll=16,
                                carry=jnp.zeros((L,), jnp.float32))
            def passb(i, carry_b):
                gsum = plsc.load_gather(scanbuf, [gbidx[pl.ds(i * L, L)]])
                inc = plsc.cumsum(gsum)
                ccbuf[pl.ds(i * L, L)] = inc - gsum + carry_b
                return carry_b + jnp.full((L,), jnp.sum(gsum), jnp.float32)

            obase = (g * RPC + r) * S
            for i in range(NSEGV):
                t = (plsc.load_gather(scanbuf, [ge_vs[i]])
                     + plsc.load_gather(ccbuf, [gq_vs[i]]))
                tstage[pl.ds(obase + i * L, L)] = t
            return carry_r

        lax.fori_loop(0, RPC, row, 0)

    @plsc.parallel_loop(0, ROWS_PER_W * NSEGV, 1, unroll=8)
    def passc(k):
        t = tstage[pl.ds(k * L, L)]
        tp = plsc.load_gather(tstage, [tpfull[pl.ds(k * L, L)]])
        iv = invbuf[pl.ds((k & (NSEGV - 1)) * L, L)]
        outstage[pl.ds(k * L, L)] = (t - tp) * iv

    pltpu.sync_copy(outstage,
                    out_hbm.at[pl.ds((b * C + c0) * S, ROWS_PER_W * S)])


@jax.jit
def _run(x, idx, cnt):
    mesh = plsc.VectorSubcoreMesh(
        core_axis_name="c", subcore_axis_name="s", num_cores=2,
        num_subcores=16)
    f = pl.kernel(
        _sc_body,
        out_type=jax.ShapeDtypeStruct((B * C * S,), jnp.float32),
        mesh=mesh,
        compiler_params=pltpu.CompilerParams(
            needs_layout_passes=False,
            skip_device_barrier=True,
            disable_bounds_checks=True,
            disable_semaphore_checks=True),
        scratch_types=[
            pltpu.VMEM((N + L,), jnp.int32),
            pltpu.VMEM((RPC * N,), jnp.float32),
            pltpu.VMEM((RPC * N,), jnp.float32),
            pltpu.VMEM((N + L,), jnp.float32),
            pltpu.VMEM((NCHUNK + L,), jnp.float32),
            pltpu.VMEM((S,), jnp.int32),
            pltpu.VMEM((S,), jnp.int32),
            pltpu.VMEM((S,), jnp.int32),
            pltpu.VMEM((S,), jnp.float32),
            pltpu.VMEM((NCHUNK,), jnp.int32),
            pltpu.VMEM((ROWS_PER_W * S + L,), jnp.float32),
            pltpu.VMEM((ROWS_PER_W * S,), jnp.int32),
            pltpu.VMEM((ROWS_PER_W * S,), jnp.float32),
            pltpu.SemaphoreType.DMA,
            pltpu.SemaphoreType.DMA,
        ],
    )
    return f(x, idx, cnt)


def kernel(input, slice_idx_mat, slice_counts):
    x = input.reshape(B * C * N)
    idx = slice_idx_mat.astype(jnp.int32).reshape(B * N)
    cnt = slice_counts.reshape(B * S).astype(jnp.float32)
    out = _run(x, idx, cnt)
    return out.reshape(B, C, S, 1)
